# baseline (speedup 1.0000x reference)
import jax
import jax.numpy as jnp
from jax import lax
from jax.experimental import pallas as pl
from jax.experimental.pallas import tpu as pltpu
from jax.experimental.pallas import tpu_sc as plsc

_TEMP = 2.0 / 3.0
_MAGIC = 0.8
_R, _N = 32, 32768
_KZ = _N - 16384
_L = 16
_U = 8
_GW = _L * _U
_GROUPS = _N // _GW


def _row_body(s_hbm, out_hbm, row_v, egrp_v, sem_in0, sem_in1, sem_in2,
              sem_in3, sem_in4, sem_in5, sem_in6, sem_in7, sem_out):
    wid = lax.axis_index("c") * 16 + lax.axis_index("s")
    iota16 = lax.iota(jnp.int32, _L)

    _SE = _N // 8
    _QE = _N // 4
    in_sems = (sem_in0, sem_in1, sem_in2, sem_in3, sem_in4, sem_in5,
               sem_in6, sem_in7)
    in_descs = [
        pltpu.async_copy(s_hbm.at[wid, pl.ds(q * _SE, _SE)],
                         row_v.at[pl.ds(q * _SE, _SE)], in_sems[q])
        for q in range(8)
    ]

    def load_bits(base):
        return lax.bitcast_convert_type(row_v[pl.ds(base, _L)], jnp.int32)

    _sc1 = jax.named_scope("p1_minmax")
    _sc1.__enter__()
    in_descs[0].wait()
    b0 = load_bits(0)
    sk, _ = plsc.sort_key_val(b0, b0)
    piv_a = sk[6]
    piv_b = jnp.maximum(sk[9], piv_a)

    mn = jnp.full((_L,), jnp.int32(2**31 - 1))
    mx = jnp.full((_L,), jnp.int32(-(2**31)))
    aca = jnp.zeros((_L,), jnp.int32)
    acb = jnp.zeros((_L,), jnp.int32)
    for q in range(8):
        if q > 0:
            in_descs[q].wait()

        @plsc.parallel_loop(q * (_SE // _L), (q + 1) * (_SE // _L),
                            unroll=_U, carry=(mn, mx, aca, acb))
        def mm_carry(i, carry):
            mn_, mx_, aca_, acb_ = carry
            b = load_bits(i * _L)
            return (jnp.minimum(mn_, b), jnp.maximum(mx_, b),
                    aca_ + plsc.all_reduce_population_count(b <= piv_a),
                    acb_ + plsc.all_reduce_population_count(b <= piv_b))

        mn, mx, aca, acb = mm_carry

    c_a = aca[0]
    c_b = acb[0]
    lo0 = mn[0]
    for i in range(1, _L):
        lo0 = jnp.minimum(lo0, mn[i])
    hi0 = mx[0]
    for i in range(1, _L):
        hi0 = jnp.maximum(hi0, mx[i])
    _sc1.__exit__(None, None, None)

    def count_le(t):
        @plsc.parallel_loop(0, _N // _L, unroll=_U,
                            carry=jnp.zeros((_L,), jnp.int32))
        def acc(i, a):
            return a + plsc.all_reduce_population_count(
                load_bits(i * _L) <= t)
        return acc[0]

    def bs_cond(st):
        return st[0] < st[1]

    def bs_body(st):
        lo, hi, c_lo, c_hi, it = st
        span_v = jnp.full((_L,), (hi - lo + 1).astype(jnp.float32))
        num_v = jnp.full((_L,), (_KZ - c_lo).astype(jnp.float32))
        den_v = jnp.full((_L,), (c_hi - c_lo).astype(jnp.float32))
        interp_p = lo - 1 + (num_v / den_v * span_v).astype(jnp.int32)[0]
        bisect_p = (lo + hi) >> 1
        p = jnp.where(lax.rem(it, jnp.int32(6)) == 5, bisect_p, interp_p)
        p = jnp.clip(p, lo, hi - 1)
        c = count_le(p)
        ge = c >= _KZ
        return (jnp.where(ge, lo, p + 1), jnp.where(ge, p, hi),
                jnp.where(ge, c_lo, c), jnp.where(ge, c, c_hi), it + 1)

    ge_a = c_a >= _KZ
    ge_b = c_b >= _KZ
    lo_i = jnp.where(ge_a, lo0, jnp.where(ge_b, piv_a + 1, piv_b + 1))
    hi_i = jnp.where(ge_a, piv_a, jnp.where(ge_b, piv_b, hi0))
    cl_i = jnp.where(ge_a, jnp.int32(0), jnp.where(ge_b, c_a, c_b))
    ch_i = jnp.where(ge_a, c_a, jnp.where(ge_b, c_b, jnp.int32(_N)))
    init = (lo_i, hi_i, cl_i, ch_i, jnp.int32(0))
    with jax.named_scope("p2_search"):
        T, _, cnt_less, _, _ = lax.while_loop(bs_cond, bs_body, init)
    r = _KZ - cnt_less

    _sc3 = jax.named_scope("p3_eq")
    _sc3.__enter__()

    @plsc.parallel_loop(0, _GROUPS, unroll=2,
                        carry=jnp.zeros((_L,), jnp.int32))
    def eq_last(g, run):
        base = g * _GW
        cnt_eq = jnp.zeros((_L,), jnp.int32)
        for u in range(_U):
            b = load_bits(base + u * _L)
            cnt_eq = cnt_eq + plsc.all_reduce_population_count(b == T)
        run = run + cnt_eq
        egrp_v[pl.ds(g * _L, _L)] = run
        return run

    _sc3.__exit__(None, None, None)
    _sc4 = jax.named_scope("p4_gscan")
    _sc4.__enter__()

    def pre(g):
        return egrp_v[pl.ds(g * _L, _L)][0]

    def gs_cond(st):
        return st[0] < st[1]

    def gs_body(st):
        glo, ghi = st
        gmid = (glo + ghi) >> 1
        ok = pre(gmid) >= r
        return jnp.where(ok, glo, gmid + 1), jnp.where(ok, gmid, ghi)

    G, _ = lax.while_loop(gs_cond, gs_body,
                          (jnp.int32(0), jnp.int32(_GROUPS - 1)))
    pre_before = jnp.where(G > 0, pre(jnp.maximum(G - 1, 0)), jnp.int32(0))
    rp = r - pre_before

    rp_v = jnp.full((_L,), rp, jnp.int32)
    prefix = jnp.zeros((_L,), jnp.int32)
    P_v = jnp.full((_L,), jnp.int32(-1))
    gbase = G * _GW
    for u in range(_U):
        b = load_bits(gbase + u * _L)
        eq = b == T
        cu = plsc.all_reduce_population_count(eq)
        found_here = (P_v < 0) & (prefix + cu >= rp_v)
        rpp = rp_v - prefix
        eq_work = eq
        for t in range(_L - 1):
            more = jnp.full((_L,), jnp.int32(t + 1)) < rpp
            f = plsc.all_reduce_ffs(eq_work)
            eq_work = eq_work & ~(more & (iota16 == f))
        P_here = plsc.all_reduce_ffs(eq_work) + (gbase + u * _L)
        P_v = jnp.where(found_here, P_here, P_v)
        prefix = prefix + cu

    _sc4.__exit__(None, None, None)
    _sc5 = jax.named_scope("p5_mask")
    _sc5.__enter__()
    Pc = lax.shift_right_logical(P_v[0], 4)
    out_descs = []
    for q in range(4):
        qlo, qhi = q * (_QE // _L), (q + 1) * (_QE // _L)
        before = Pc >= qhi
        after = Pc < qlo

        @pl.when(before)
        def _():
            @plsc.parallel_loop(qlo, qhi, unroll=_U)
            def mask_le(i):
                v = row_v[pl.ds(i * _L, _L)]
                b = lax.bitcast_convert_type(v, jnp.int32)
                row_v[pl.ds(i * _L, _L)] = jnp.where(b <= T, 0.0, v)

        @pl.when(after)
        def _():
            @plsc.parallel_loop(qlo, qhi, unroll=_U)
            def mask_lt(i):
                v = row_v[pl.ds(i * _L, _L)]
                b = lax.bitcast_convert_type(v, jnp.int32)
                row_v[pl.ds(i * _L, _L)] = jnp.where(b < T, 0.0, v)

        @pl.when(jnp.logical_not(before | after))
        def _():
            @plsc.parallel_loop(qlo, qhi, unroll=_U)
            def mask_full(i):
                base = i * _L
                v = row_v[pl.ds(base, _L)]
                b = lax.bitcast_convert_type(v, jnp.int32)
                idx = iota16 + base
                zero = (b < T) | ((b == T) & (idx <= P_v))
                row_v[pl.ds(base, _L)] = jnp.where(zero, 0.0, v)

        out_descs.append(
            pltpu.async_copy(row_v.at[pl.ds(q * _QE, _QE)],
                             out_hbm.at[wid, pl.ds(q * _QE, _QE)], sem_out))
    for d in out_descs:
        d.wait()
    _sc5.__exit__(None, None, None)


def _sc_select(s):
    kfn = pl.kernel(
        _row_body,
        out_type=jax.ShapeDtypeStruct((_R, _N), jnp.float32),
        mesh=plsc.VectorSubcoreMesh(
            core_axis_name="c", subcore_axis_name="s",
            num_cores=2, num_subcores=16),
        scratch_types=[
            pltpu.VMEM((_N,), jnp.float32),
            pltpu.VMEM((_GROUPS * _L,), jnp.int32),
            pltpu.SemaphoreType.DMA,
            pltpu.SemaphoreType.DMA,
            pltpu.SemaphoreType.DMA,
            pltpu.SemaphoreType.DMA,
            pltpu.SemaphoreType.DMA,
            pltpu.SemaphoreType.DMA,
            pltpu.SemaphoreType.DMA,
            pltpu.SemaphoreType.DMA,
            pltpu.SemaphoreType.DMA,
        ],
        compiler_params=pltpu.CompilerParams(needs_layout_passes=False),
    )
    return kfn(s)


def kernel(z_loga):
    z2 = z_loga.reshape(-1, z_loga.shape[-1])
    s = jax.nn.sigmoid(z2 / _TEMP * _MAGIC)
    return _sc_select(s).reshape(_R, _N)


# speedup vs baseline: 1.0001x; 1.0001x over previous
import jax
import jax.numpy as jnp
from jax import lax
from jax.experimental import pallas as pl
from jax.experimental.pallas import tpu as pltpu
from jax.experimental.pallas import tpu_sc as plsc

_TEMP = 2.0 / 3.0
_MAGIC = 0.8
_R, _N = 32, 32768
_KZ = _N - 16384
_L = 16
_U = 8
_GW = _L * _U
_GROUPS = _N // _GW


def _row_body(s_hbm, out_hbm, row_v, egrp_v, sem_in0, sem_in1, sem_in2,
              sem_in3, sem_in4, sem_in5, sem_in6, sem_in7, sem_out):
    wid = lax.axis_index("c") * 16 + lax.axis_index("s")
    iota16 = lax.iota(jnp.int32, _L)

    _SE = _N // 8
    _QE = _N // 4
    in_sems = (sem_in0, sem_in1, sem_in2, sem_in3, sem_in4, sem_in5,
               sem_in6, sem_in7)
    in_descs = [
        pltpu.async_copy(s_hbm.at[wid, pl.ds(q * _SE, _SE)],
                         row_v.at[pl.ds(q * _SE, _SE)], in_sems[q])
        for q in range(8)
    ]

    def load_bits(base):
        return lax.bitcast_convert_type(row_v[pl.ds(base, _L)], jnp.int32)

    _sc1 = jax.named_scope("p1_minmax")
    _sc1.__enter__()
    in_descs[0].wait()
    b0 = load_bits(0)
    sk, _ = plsc.sort_key_val(b0, b0)
    piv_a = sk[6]
    piv_b = jnp.maximum(sk[9], piv_a)

    mn = jnp.full((_L,), jnp.int32(2**31 - 1))
    mx = jnp.full((_L,), jnp.int32(-(2**31)))
    aca = jnp.zeros((_L,), jnp.int32)
    acb = jnp.zeros((_L,), jnp.int32)
    for q in range(8):
        if q > 0:
            in_descs[q].wait()

        @plsc.parallel_loop(q * (_SE // _L), (q + 1) * (_SE // _L),
                            unroll=_U, carry=(mn, mx, aca, acb))
        def mm_carry(i, carry):
            mn_, mx_, aca_, acb_ = carry
            b = load_bits(i * _L)
            return (jnp.minimum(mn_, b), jnp.maximum(mx_, b),
                    aca_ + (b <= piv_a).astype(jnp.int32),
                    acb_ + (b <= piv_b).astype(jnp.int32))

        mn, mx, aca, acb = mm_carry

    c_a = aca[0]
    c_b = acb[0]
    for i in range(1, _L):
        c_a = c_a + aca[i]
        c_b = c_b + acb[i]
    lo0 = mn[0]
    for i in range(1, _L):
        lo0 = jnp.minimum(lo0, mn[i])
    hi0 = mx[0]
    for i in range(1, _L):
        hi0 = jnp.maximum(hi0, mx[i])
    _sc1.__exit__(None, None, None)

    def count_le(t):
        @plsc.parallel_loop(0, _N // _L, unroll=_U,
                            carry=jnp.zeros((_L,), jnp.int32))
        def acc(i, a):
            return a + plsc.all_reduce_population_count(
                load_bits(i * _L) <= t)
        return acc[0]

    def bs_cond(st):
        return st[0] < st[1]

    def bs_body(st):
        lo, hi, c_lo, c_hi, it = st
        span_v = jnp.full((_L,), (hi - lo + 1).astype(jnp.float32))
        num_v = jnp.full((_L,), (_KZ - c_lo).astype(jnp.float32))
        den_v = jnp.full((_L,), (c_hi - c_lo).astype(jnp.float32))
        interp_p = lo - 1 + (num_v / den_v * span_v).astype(jnp.int32)[0]
        bisect_p = (lo + hi) >> 1
        p = jnp.where(lax.rem(it, jnp.int32(6)) == 5, bisect_p, interp_p)
        p = jnp.clip(p, lo, hi - 1)
        c = count_le(p)
        ge = c >= _KZ
        return (jnp.where(ge, lo, p + 1), jnp.where(ge, p, hi),
                jnp.where(ge, c_lo, c), jnp.where(ge, c, c_hi), it + 1)

    ge_a = c_a >= _KZ
    ge_b = c_b >= _KZ
    lo_i = jnp.where(ge_a, lo0, jnp.where(ge_b, piv_a + 1, piv_b + 1))
    hi_i = jnp.where(ge_a, piv_a, jnp.where(ge_b, piv_b, hi0))
    cl_i = jnp.where(ge_a, jnp.int32(0), jnp.where(ge_b, c_a, c_b))
    ch_i = jnp.where(ge_a, c_a, jnp.where(ge_b, c_b, jnp.int32(_N)))
    init = (lo_i, hi_i, cl_i, ch_i, jnp.int32(0))
    with jax.named_scope("p2_search"):
        T, _, cnt_less, _, _ = lax.while_loop(bs_cond, bs_body, init)
    r = _KZ - cnt_less

    _sc3 = jax.named_scope("p3_eq")
    _sc3.__enter__()

    @plsc.parallel_loop(0, _GROUPS, unroll=2,
                        carry=jnp.zeros((_L,), jnp.int32))
    def eq_last(g, run):
        base = g * _GW
        cnt_eq = jnp.zeros((_L,), jnp.int32)
        for u in range(_U):
            b = load_bits(base + u * _L)
            cnt_eq = cnt_eq + plsc.all_reduce_population_count(b == T)
        run = run + cnt_eq
        egrp_v[pl.ds(g * _L, _L)] = run
        return run

    _sc3.__exit__(None, None, None)
    _sc4 = jax.named_scope("p4_gscan")
    _sc4.__enter__()

    def pre(g):
        return egrp_v[pl.ds(g * _L, _L)][0]

    def gs_cond(st):
        return st[0] < st[1]

    def gs_body(st):
        glo, ghi = st
        gmid = (glo + ghi) >> 1
        ok = pre(gmid) >= r
        return jnp.where(ok, glo, gmid + 1), jnp.where(ok, gmid, ghi)

    G, _ = lax.while_loop(gs_cond, gs_body,
                          (jnp.int32(0), jnp.int32(_GROUPS - 1)))
    pre_before = jnp.where(G > 0, pre(jnp.maximum(G - 1, 0)), jnp.int32(0))
    rp = r - pre_before

    rp_v = jnp.full((_L,), rp, jnp.int32)
    prefix = jnp.zeros((_L,), jnp.int32)
    P_v = jnp.full((_L,), jnp.int32(-1))
    gbase = G * _GW
    for u in range(_U):
        b = load_bits(gbase + u * _L)
        eq = b == T
        cu = plsc.all_reduce_population_count(eq)
        found_here = (P_v < 0) & (prefix + cu >= rp_v)
        rpp = rp_v - prefix
        eq_work = eq
        for t in range(_L - 1):
            more = jnp.full((_L,), jnp.int32(t + 1)) < rpp
            f = plsc.all_reduce_ffs(eq_work)
            eq_work = eq_work & ~(more & (iota16 == f))
        P_here = plsc.all_reduce_ffs(eq_work) + (gbase + u * _L)
        P_v = jnp.where(found_here, P_here, P_v)
        prefix = prefix + cu

    _sc4.__exit__(None, None, None)
    _sc5 = jax.named_scope("p5_mask")
    _sc5.__enter__()
    Pc = lax.shift_right_logical(P_v[0], 4)
    out_descs = []
    for q in range(4):
        qlo, qhi = q * (_QE // _L), (q + 1) * (_QE // _L)
        before = Pc >= qhi
        after = Pc < qlo

        @pl.when(before)
        def _():
            @plsc.parallel_loop(qlo, qhi, unroll=_U)
            def mask_le(i):
                v = row_v[pl.ds(i * _L, _L)]
                b = lax.bitcast_convert_type(v, jnp.int32)
                row_v[pl.ds(i * _L, _L)] = jnp.where(b <= T, 0.0, v)

        @pl.when(after)
        def _():
            @plsc.parallel_loop(qlo, qhi, unroll=_U)
            def mask_lt(i):
                v = row_v[pl.ds(i * _L, _L)]
                b = lax.bitcast_convert_type(v, jnp.int32)
                row_v[pl.ds(i * _L, _L)] = jnp.where(b < T, 0.0, v)

        @pl.when(jnp.logical_not(before | after))
        def _():
            @plsc.parallel_loop(qlo, qhi, unroll=_U)
            def mask_full(i):
                base = i * _L
                v = row_v[pl.ds(base, _L)]
                b = lax.bitcast_convert_type(v, jnp.int32)
                idx = iota16 + base
                zero = (b < T) | ((b == T) & (idx <= P_v))
                row_v[pl.ds(base, _L)] = jnp.where(zero, 0.0, v)

        out_descs.append(
            pltpu.async_copy(row_v.at[pl.ds(q * _QE, _QE)],
                             out_hbm.at[wid, pl.ds(q * _QE, _QE)], sem_out))
    for d in out_descs:
        d.wait()
    _sc5.__exit__(None, None, None)


def _sc_select(s):
    kfn = pl.kernel(
        _row_body,
        out_type=jax.ShapeDtypeStruct((_R, _N), jnp.float32),
        mesh=plsc.VectorSubcoreMesh(
            core_axis_name="c", subcore_axis_name="s",
            num_cores=2, num_subcores=16),
        scratch_types=[
            pltpu.VMEM((_N,), jnp.float32),
            pltpu.VMEM((_GROUPS * _L,), jnp.int32),
            pltpu.SemaphoreType.DMA,
            pltpu.SemaphoreType.DMA,
            pltpu.SemaphoreType.DMA,
            pltpu.SemaphoreType.DMA,
            pltpu.SemaphoreType.DMA,
            pltpu.SemaphoreType.DMA,
            pltpu.SemaphoreType.DMA,
            pltpu.SemaphoreType.DMA,
            pltpu.SemaphoreType.DMA,
        ],
        compiler_params=pltpu.CompilerParams(needs_layout_passes=False),
    )
    return kfn(s)


def kernel(z_loga):
    z2 = z_loga.reshape(-1, z_loga.shape[-1])
    s = jax.nn.sigmoid(z2 / _TEMP * _MAGIC)
    return _sc_select(s).reshape(_R, _N)


# speedup vs baseline: 1.0347x; 1.0345x over previous
import jax
import jax.numpy as jnp
from jax import lax
from jax.experimental import pallas as pl
from jax.experimental.pallas import tpu as pltpu
from jax.experimental.pallas import tpu_sc as plsc

_TEMP = 2.0 / 3.0
_MAGIC = 0.8
_R, _N = 32, 32768
_KZ = _N - 16384
_L = 16
_U = 8
_GW = _L * _U
_GROUPS = _N // _GW


def _row_body(s_hbm, out_hbm, row_v, egrp_v, sem_in0, sem_in1, sem_in2,
              sem_in3, sem_in4, sem_in5, sem_in6, sem_in7, sem_out):
    wid = lax.axis_index("c") * 16 + lax.axis_index("s")
    iota16 = lax.iota(jnp.int32, _L)

    _SE = _N // 8
    _QE = _N // 4
    in_sems = (sem_in0, sem_in1, sem_in2, sem_in3, sem_in4, sem_in5,
               sem_in6, sem_in7)
    in_descs = [
        pltpu.async_copy(s_hbm.at[wid, pl.ds(q * _SE, _SE)],
                         row_v.at[pl.ds(q * _SE, _SE)], in_sems[q])
        for q in range(8)
    ]

    def load_bits(base):
        return lax.bitcast_convert_type(row_v[pl.ds(base, _L)], jnp.int32)

    _sc1 = jax.named_scope("p1_minmax")
    _sc1.__enter__()
    in_descs[0].wait()
    b0 = load_bits(0)
    sk, _ = plsc.sort_key_val(b0, b0)
    piv_a = sk[_L // 2 - 1]

    mn = jnp.full((_L,), jnp.int32(2**31 - 1))
    mx = jnp.full((_L,), jnp.int32(-(2**31)))
    aca = jnp.zeros((_L,), jnp.int32)
    for q in range(8):
        if q > 0:
            in_descs[q].wait()

        @plsc.parallel_loop(q * (_SE // _L), (q + 1) * (_SE // _L),
                            unroll=_U, carry=(mn, mx, aca))
        def mm_carry(i, carry):
            mn_, mx_, aca_ = carry
            b = load_bits(i * _L)
            return (jnp.minimum(mn_, b), jnp.maximum(mx_, b),
                    aca_ + (b <= piv_a).astype(jnp.int32))

        mn, mx, aca = mm_carry

    c_a = aca[0]
    for i in range(1, _L):
        c_a = c_a + aca[i]
    lo0 = mn[0]
    for i in range(1, _L):
        lo0 = jnp.minimum(lo0, mn[i])
    hi0 = mx[0]
    for i in range(1, _L):
        hi0 = jnp.maximum(hi0, mx[i])
    _sc1.__exit__(None, None, None)

    def count_le(t):
        @plsc.parallel_loop(0, _N // _L, unroll=_U,
                            carry=jnp.zeros((_L,), jnp.int32))
        def acc(i, a):
            return a + plsc.all_reduce_population_count(
                load_bits(i * _L) <= t)
        return acc[0]

    def bs_cond(st):
        return st[0] < st[1]

    def bs_body(st):
        lo, hi, c_lo, c_hi, it = st
        span_v = jnp.full((_L,), (hi - lo + 1).astype(jnp.float32))
        num_v = jnp.full((_L,), (_KZ - c_lo).astype(jnp.float32))
        den_v = jnp.full((_L,), (c_hi - c_lo).astype(jnp.float32))
        interp_p = lo - 1 + (num_v / den_v * span_v).astype(jnp.int32)[0]
        bisect_p = (lo + hi) >> 1
        p = jnp.where(lax.rem(it, jnp.int32(6)) == 5, bisect_p, interp_p)
        p = jnp.clip(p, lo, hi - 1)
        c = count_le(p)
        ge = c >= _KZ
        return (jnp.where(ge, lo, p + 1), jnp.where(ge, p, hi),
                jnp.where(ge, c_lo, c), jnp.where(ge, c, c_hi), it + 1)

    ge_a = c_a >= _KZ
    init = (jnp.where(ge_a, lo0, piv_a + 1), jnp.where(ge_a, piv_a, hi0),
            jnp.where(ge_a, jnp.int32(0), c_a),
            jnp.where(ge_a, c_a, jnp.int32(_N)), jnp.int32(0))
    with jax.named_scope("p2_search"):
        T, _, cnt_less, _, _ = lax.while_loop(bs_cond, bs_body, init)
    r = _KZ - cnt_less

    _sc3 = jax.named_scope("p3_eq")
    _sc3.__enter__()

    @plsc.parallel_loop(0, _GROUPS, unroll=2,
                        carry=jnp.zeros((_L,), jnp.int32))
    def eq_last(g, run):
        base = g * _GW
        cnt_eq = jnp.zeros((_L,), jnp.int32)
        for u in range(_U):
            b = load_bits(base + u * _L)
            cnt_eq = cnt_eq + plsc.all_reduce_population_count(b == T)
        run = run + cnt_eq
        egrp_v[pl.ds(g * _L, _L)] = run
        return run

    _sc3.__exit__(None, None, None)
    _sc4 = jax.named_scope("p4_gscan")
    _sc4.__enter__()

    def pre(g):
        return egrp_v[pl.ds(g * _L, _L)][0]

    def gs_cond(st):
        return st[0] < st[1]

    def gs_body(st):
        glo, ghi = st
        gmid = (glo + ghi) >> 1
        ok = pre(gmid) >= r
        return jnp.where(ok, glo, gmid + 1), jnp.where(ok, gmid, ghi)

    G, _ = lax.while_loop(gs_cond, gs_body,
                          (jnp.int32(0), jnp.int32(_GROUPS - 1)))
    pre_before = jnp.where(G > 0, pre(jnp.maximum(G - 1, 0)), jnp.int32(0))
    rp = r - pre_before

    rp_v = jnp.full((_L,), rp, jnp.int32)
    prefix = jnp.zeros((_L,), jnp.int32)
    P_v = jnp.full((_L,), jnp.int32(-1))
    gbase = G * _GW
    for u in range(_U):
        b = load_bits(gbase + u * _L)
        eq = b == T
        cu = plsc.all_reduce_population_count(eq)
        found_here = (P_v < 0) & (prefix + cu >= rp_v)
        rpp = rp_v - prefix
        eq_work = eq
        for t in range(_L - 1):
            more = jnp.full((_L,), jnp.int32(t + 1)) < rpp
            f = plsc.all_reduce_ffs(eq_work)
            eq_work = eq_work & ~(more & (iota16 == f))
        P_here = plsc.all_reduce_ffs(eq_work) + (gbase + u * _L)
        P_v = jnp.where(found_here, P_here, P_v)
        prefix = prefix + cu

    _sc4.__exit__(None, None, None)
    _sc5 = jax.named_scope("p5_mask")
    _sc5.__enter__()
    Pc = lax.shift_right_logical(P_v[0], 4)
    out_descs = []
    for q in range(4):
        qlo, qhi = q * (_QE // _L), (q + 1) * (_QE // _L)
        before = Pc >= qhi
        after = Pc < qlo

        @pl.when(before)
        def _():
            @plsc.parallel_loop(qlo, qhi, unroll=_U)
            def mask_le(i):
                v = row_v[pl.ds(i * _L, _L)]
                b = lax.bitcast_convert_type(v, jnp.int32)
                row_v[pl.ds(i * _L, _L)] = jnp.where(b <= T, 0.0, v)

        @pl.when(after)
        def _():
            @plsc.parallel_loop(qlo, qhi, unroll=_U)
            def mask_lt(i):
                v = row_v[pl.ds(i * _L, _L)]
                b = lax.bitcast_convert_type(v, jnp.int32)
                row_v[pl.ds(i * _L, _L)] = jnp.where(b < T, 0.0, v)

        @pl.when(jnp.logical_not(before | after))
        def _():
            @plsc.parallel_loop(qlo, qhi, unroll=_U)
            def mask_full(i):
                base = i * _L
                v = row_v[pl.ds(base, _L)]
                b = lax.bitcast_convert_type(v, jnp.int32)
                idx = iota16 + base
                zero = (b < T) | ((b == T) & (idx <= P_v))
                row_v[pl.ds(base, _L)] = jnp.where(zero, 0.0, v)

        out_descs.append(
            pltpu.async_copy(row_v.at[pl.ds(q * _QE, _QE)],
                             out_hbm.at[wid, pl.ds(q * _QE, _QE)], sem_out))
    for d in out_descs:
        d.wait()
    _sc5.__exit__(None, None, None)


def _sc_select(s):
    kfn = pl.kernel(
        _row_body,
        out_type=jax.ShapeDtypeStruct((_R, _N), jnp.float32),
        mesh=plsc.VectorSubcoreMesh(
            core_axis_name="c", subcore_axis_name="s",
            num_cores=2, num_subcores=16),
        scratch_types=[
            pltpu.VMEM((_N,), jnp.float32),
            pltpu.VMEM((_GROUPS * _L,), jnp.int32),
            pltpu.SemaphoreType.DMA,
            pltpu.SemaphoreType.DMA,
            pltpu.SemaphoreType.DMA,
            pltpu.SemaphoreType.DMA,
            pltpu.SemaphoreType.DMA,
            pltpu.SemaphoreType.DMA,
            pltpu.SemaphoreType.DMA,
            pltpu.SemaphoreType.DMA,
            pltpu.SemaphoreType.DMA,
        ],
        compiler_params=pltpu.CompilerParams(needs_layout_passes=False),
    )
    return kfn(s)


def kernel(z_loga):
    z2 = z_loga.reshape(-1, z_loga.shape[-1])
    s = jax.nn.sigmoid(z2 / _TEMP * _MAGIC)
    return _sc_select(s).reshape(_R, _N)
